# baseline (speedup 1.0000x reference)
import functools

import jax
import jax.numpy as jnp
from jax import lax
from jax.experimental import pallas as pl
from jax.experimental.pallas import tpu as pltpu
from jax.experimental.pallas import tpu_sc as plsc

B, T, V, H = 1024, 50, 1000, 1024
L = 16
VP = 1024


def _make_counts_kernel():
    info = plsc.get_sparse_core_info()
    nc, ns = info.num_cores, info.num_subcores
    nw = nc * ns
    bpw = B // nw

    mesh = plsc.VectorSubcoreMesh(core_axis_name="c", subcore_axis_name="s")

    @functools.partial(
        pl.kernel,
        mesh=mesh,
        compiler_params=pltpu.CompilerParams(
            needs_layout_passes=False,
            use_tc_tiling_on_sc=False,
            skip_device_barrier=True,
        ),
        out_type=jax.ShapeDtypeStruct((nw, bpw * VP), jnp.float32),
        scratch_types=[
            pltpu.VMEM((bpw, T), jnp.int32),
            pltpu.VMEM((bpw * VP,), jnp.float32),
            pltpu.SemaphoreType.DMA,
            pltpu.SemaphoreType.DMA,
        ],
    )
    def k(idx_hbm, cnt_hbm, idx_v, cnt_v, sem, sem_in):
        wid = lax.axis_index("s") * nc + lax.axis_index("c")
        idx_cp = pltpu.async_copy(idx_hbm.at[pl.ds(wid * bpw, bpw)], idx_v,
                                  sem_in)

        zeros = jnp.zeros((L,), jnp.float32)

        def zloop(i, _):
            for u in range(16):
                cnt_v[pl.ds((i * 16 + u) * L, L)] = zeros
            return 0

        lax.fori_loop(0, bpw * VP // (16 * L), zloop, 0)
        idx_cp.wait()

        ones = jnp.ones((L,), jnp.float32)
        lane = lax.iota(jnp.int32, L)
        nfull = T // L
        tail_mask = lane >= (nfull + 1) * L - T

        def sloop(r, _):
            rbase = r * VP
            for g in range(nfull):
                addr = idx_v[r, pl.ds(g * L, L)] + rbase
                plsc.addupdate_scatter(cnt_v, [addr], ones)
            if T % L:
                addr = idx_v[r, pl.ds(T - L, L)] + rbase
                plsc.addupdate_scatter(cnt_v, [addr], ones, mask=tail_mask)
            return 0

        half = bpw // 2 * VP
        lax.fori_loop(0, bpw // 2, sloop, 0)
        first = pltpu.async_copy(cnt_v.at[pl.ds(0, half)],
                                 cnt_hbm.at[wid, pl.ds(0, half)], sem)
        lax.fori_loop(bpw // 2, bpw, sloop, 0)
        first.wait()
        pltpu.sync_copy(cnt_v.at[pl.ds(half, half)],
                        cnt_hbm.at[wid, pl.ds(half, half)])

    return k


_counts_kernel = _make_counts_kernel()


def _mm_body(a_ref, b_ref, o_ref):
    a = a_ref[...].reshape(_BM, VP)
    o_ref[...] = jnp.dot(a[:, :V].astype(jnp.bfloat16), b_ref[...],
                         preferred_element_type=jnp.float32)


_BM = 1024
_matmul = pl.pallas_call(
    _mm_body,
    grid=(B // _BM,),
    in_specs=[
        pl.BlockSpec((_BM // 8, 8, VP // 128, 128), lambda i: (i, 0, 0, 0)),
        pl.BlockSpec((V, H), lambda i: (0, 0)),
    ],
    out_specs=pl.BlockSpec((_BM, H), lambda i: (i, 0)),
    out_shape=jax.ShapeDtypeStruct((B, H), jnp.float32),
)


def kernel(indices, codebook):
    counts = _counts_kernel(indices.astype(jnp.int32))
    counts4 = counts.reshape(B // 8, 8, VP // 128, 128)
    return _matmul(counts4, codebook.astype(jnp.bfloat16))


# speedup vs baseline: 1.0296x; 1.0296x over previous
import functools

import jax
import jax.numpy as jnp
from jax import lax
from jax.experimental import pallas as pl
from jax.experimental.pallas import tpu as pltpu
from jax.experimental.pallas import tpu_sc as plsc

B, T, V, H = 1024, 50, 1000, 1024
L = 16
VP = 1024


def _make_counts_kernel():
    info = plsc.get_sparse_core_info()
    nc, ns = info.num_cores, info.num_subcores
    nw = nc * ns
    bpw = B // nw

    mesh = plsc.VectorSubcoreMesh(core_axis_name="c", subcore_axis_name="s")

    @functools.partial(
        pl.kernel,
        mesh=mesh,
        compiler_params=pltpu.CompilerParams(
            needs_layout_passes=False,
            use_tc_tiling_on_sc=False,
        ),
        out_type=jax.ShapeDtypeStruct((nw, bpw * VP), jnp.float32),
        scratch_types=[
            pltpu.VMEM((bpw, T), jnp.int32),
            pltpu.VMEM((bpw * VP,), jnp.float32),
            pltpu.SemaphoreType.DMA,
            pltpu.SemaphoreType.DMA,
        ],
    )
    def k(idx_hbm, cnt_hbm, idx_v, cnt_v, sem, sem_in):
        wid = lax.axis_index("s") * nc + lax.axis_index("c")
        idx_cp = pltpu.async_copy(idx_hbm.at[pl.ds(wid * bpw, bpw)], idx_v,
                                  sem_in)

        zeros = jnp.zeros((L,), jnp.float32)

        def zloop(i, _):
            for u in range(16):
                cnt_v[pl.ds((i * 16 + u) * L, L)] = zeros
            return 0

        lax.fori_loop(0, bpw * VP // (16 * L), zloop, 0)
        idx_cp.wait()

        ones = jnp.ones((L,), jnp.float32)
        lane = lax.iota(jnp.int32, L)
        nfull = T // L
        tail_mask = lane >= (nfull + 1) * L - T

        def sloop(r, _):
            rbase = r * VP
            for g in range(nfull):
                addr = idx_v[r, pl.ds(g * L, L)] + rbase
                plsc.addupdate_scatter(cnt_v, [addr], ones)
            if T % L:
                addr = idx_v[r, pl.ds(T - L, L)] + rbase
                plsc.addupdate_scatter(cnt_v, [addr], ones, mask=tail_mask)
            return 0

        half = bpw // 2 * VP
        lax.fori_loop(0, bpw // 2, sloop, 0)
        first = pltpu.async_copy(cnt_v.at[pl.ds(0, half)],
                                 cnt_hbm.at[wid, pl.ds(0, half)], sem)
        lax.fori_loop(bpw // 2, bpw, sloop, 0)
        first.wait()
        pltpu.sync_copy(cnt_v.at[pl.ds(half, half)],
                        cnt_hbm.at[wid, pl.ds(half, half)])

    return k


_counts_kernel = _make_counts_kernel()


def _mm_body(a_ref, b_ref, o_ref):
    a = a_ref[...].reshape(_BM, VP)
    o_ref[...] = jnp.dot(a[:, :V].astype(jnp.bfloat16), b_ref[...],
                         preferred_element_type=jnp.float32)


_BM = 512
_matmul = pl.pallas_call(
    _mm_body,
    grid=(B // _BM,),
    in_specs=[
        pl.BlockSpec((_BM // 8, 8, VP // 128, 128), lambda i: (i, 0, 0, 0)),
        pl.BlockSpec((V, H), lambda i: (0, 0)),
    ],
    out_specs=pl.BlockSpec((_BM, H), lambda i: (i, 0)),
    out_shape=jax.ShapeDtypeStruct((B, H), jnp.float32),
)


def kernel(indices, codebook):
    counts = _counts_kernel(indices.astype(jnp.int32))
    counts4 = counts.reshape(B // 8, 8, VP // 128, 128)
    return _matmul(counts4, codebook.astype(jnp.bfloat16))


# speedup vs baseline: 1.0322x; 1.0025x over previous
import functools

import jax
import jax.numpy as jnp
from jax import lax
from jax.experimental import pallas as pl
from jax.experimental.pallas import tpu as pltpu
from jax.experimental.pallas import tpu_sc as plsc

B, T, V, H = 1024, 50, 1000, 1024
L = 16
VP = 1024


def _make_counts_kernel():
    info = plsc.get_sparse_core_info()
    nc, ns = info.num_cores, info.num_subcores
    nw = nc * ns
    bpw = B // nw

    mesh = plsc.VectorSubcoreMesh(core_axis_name="c", subcore_axis_name="s")

    @functools.partial(
        pl.kernel,
        mesh=mesh,
        compiler_params=pltpu.CompilerParams(
            needs_layout_passes=False,
            use_tc_tiling_on_sc=False,
        ),
        out_type=jax.ShapeDtypeStruct((nw, bpw * VP), jnp.float32),
        scratch_types=[
            pltpu.VMEM((bpw * T,), jnp.int32),
            pltpu.VMEM((bpw * VP,), jnp.float32),
            pltpu.SemaphoreType.DMA,
            pltpu.SemaphoreType.DMA,
        ],
    )
    def k(idx_hbm, cnt_hbm, idx_v, cnt_v, sem, sem_in):
        wid = lax.axis_index("s") * nc + lax.axis_index("c")
        idx_cp = pltpu.async_copy(idx_hbm.at[pl.ds(wid * (bpw * T), bpw * T)],
                                  idx_v, sem_in)

        zeros = jnp.zeros((L,), jnp.float32)

        def zloop(i, _):
            for u in range(16):
                cnt_v[pl.ds((i * 16 + u) * L, L)] = zeros
            return 0

        lax.fori_loop(0, bpw * VP // (16 * L), zloop, 0)
        idx_cp.wait()

        ones = jnp.ones((L,), jnp.float32)
        lane = lax.iota(jnp.int32, L)
        nfull = T // L
        tail_mask = lane >= (nfull + 1) * L - T

        def sloop(r, _):
            rbase = r * VP
            for g in range(nfull):
                addr = idx_v[pl.ds(r * T + g * L, L)] + rbase
                plsc.addupdate_scatter(cnt_v, [addr], ones)
            if T % L:
                addr = idx_v[pl.ds(r * T + T - L, L)] + rbase
                plsc.addupdate_scatter(cnt_v, [addr], ones, mask=tail_mask)
            return 0

        half = bpw // 2 * VP
        lax.fori_loop(0, bpw // 2, sloop, 0)
        first = pltpu.async_copy(cnt_v.at[pl.ds(0, half)],
                                 cnt_hbm.at[wid, pl.ds(0, half)], sem)
        lax.fori_loop(bpw // 2, bpw, sloop, 0)
        first.wait()
        pltpu.sync_copy(cnt_v.at[pl.ds(half, half)],
                        cnt_hbm.at[wid, pl.ds(half, half)])

    return k


_counts_kernel = _make_counts_kernel()


def _mm_body(a_ref, b_ref, o_ref):
    a = a_ref[...].reshape(_BM, VP)
    o_ref[...] = jnp.dot(a[:, :V].astype(jnp.bfloat16), b_ref[...],
                         preferred_element_type=jnp.float32)


_BM = 512
_matmul = pl.pallas_call(
    _mm_body,
    grid=(B // _BM,),
    in_specs=[
        pl.BlockSpec((_BM // 8, 8, VP // 128, 128), lambda i: (i, 0, 0, 0)),
        pl.BlockSpec((V, H), lambda i: (0, 0)),
    ],
    out_specs=pl.BlockSpec((_BM, H), lambda i: (i, 0)),
    out_shape=jax.ShapeDtypeStruct((B, H), jnp.float32),
)


def kernel(indices, codebook):
    counts = _counts_kernel(indices.astype(jnp.int32).reshape(B * T))
    counts4 = counts.reshape(B // 8, 8, VP // 128, 128)
    return _matmul(counts4, codebook.astype(jnp.bfloat16))


# speedup vs baseline: 1.0338x; 1.0015x over previous
import functools

import jax
import jax.numpy as jnp
from jax import lax
from jax.experimental import pallas as pl
from jax.experimental.pallas import tpu as pltpu
from jax.experimental.pallas import tpu_sc as plsc

B, T, V, H = 1024, 50, 1000, 1024
L = 16
VP = 1024


def _make_counts_kernel():
    info = plsc.get_sparse_core_info()
    nc, ns = info.num_cores, info.num_subcores
    nw = nc * ns
    bpw = B // nw

    mesh = plsc.VectorSubcoreMesh(core_axis_name="c", subcore_axis_name="s")

    @functools.partial(
        pl.kernel,
        mesh=mesh,
        compiler_params=pltpu.CompilerParams(
            needs_layout_passes=False,
            use_tc_tiling_on_sc=False,
        ),
        out_type=jax.ShapeDtypeStruct((nw, bpw * VP), jnp.float32),
        scratch_types=[
            pltpu.VMEM((bpw * T,), jnp.int32),
            pltpu.VMEM((bpw * VP,), jnp.float32),
            pltpu.SemaphoreType.DMA,
            pltpu.SemaphoreType.DMA,
        ],
    )
    def k(idx_hbm, cnt_hbm, idx_v, cnt_v, sem, sem_in):
        wid = lax.axis_index("s") * nc + lax.axis_index("c")
        idx_cp = pltpu.async_copy(idx_hbm.at[pl.ds(wid * (bpw * T), bpw * T)],
                                  idx_v, sem_in)

        zeros = jnp.zeros((L,), jnp.float32)

        def zloop(i, _):
            for u in range(16):
                cnt_v[pl.ds((i * 16 + u) * L, L)] = zeros
            return 0

        lax.fori_loop(0, bpw * VP // (16 * L), zloop, 0)
        idx_cp.wait()

        ones = jnp.ones((L,), jnp.float32)
        lane = lax.iota(jnp.int32, L)
        nfull = T // L
        tail_mask = lane >= (nfull + 1) * L - T

        def sloop(r, _):
            rbase = r * VP
            for g in range(nfull):
                addr = idx_v[pl.ds(r * T + g * L, L)] + rbase
                plsc.addupdate_scatter(cnt_v, [addr], ones)
            if T % L:
                addr = idx_v[pl.ds(r * T + T - L, L)] + rbase
                plsc.addupdate_scatter(cnt_v, [addr], ones, mask=tail_mask)
            return 0

        nq = 4
        qrows = bpw // nq
        qw = qrows * VP
        for q in range(nq):
            lax.fori_loop(q * qrows, (q + 1) * qrows, sloop, 0)
            pltpu.async_copy(cnt_v.at[pl.ds(q * qw, qw)],
                             cnt_hbm.at[wid, pl.ds(q * qw, qw)], sem)
        for q in range(nq):
            pltpu.make_async_copy(cnt_v.at[pl.ds(q * qw, qw)],
                                  cnt_hbm.at[wid, pl.ds(q * qw, qw)],
                                  sem).wait()

    return k


_counts_kernel = _make_counts_kernel()


def _mm_body(a_ref, b_ref, o_ref):
    a = a_ref[...].reshape(_BM, VP)
    o_ref[...] = jnp.dot(a[:, :V].astype(jnp.bfloat16), b_ref[...],
                         preferred_element_type=jnp.float32)


_BM = 512
_matmul = pl.pallas_call(
    _mm_body,
    grid=(B // _BM,),
    in_specs=[
        pl.BlockSpec((_BM // 8, 8, VP // 128, 128), lambda i: (i, 0, 0, 0)),
        pl.BlockSpec((V, H), lambda i: (0, 0)),
    ],
    out_specs=pl.BlockSpec((_BM, H), lambda i: (i, 0)),
    out_shape=jax.ShapeDtypeStruct((B, H), jnp.float32),
)


def kernel(indices, codebook):
    counts = _counts_kernel(indices.astype(jnp.int32).reshape(B * T))
    counts4 = counts.reshape(B // 8, 8, VP // 128, 128)
    return _matmul(counts4, codebook.astype(jnp.bfloat16))
